# baseline (speedup 1.0000x reference)
import functools
import jax
import jax.numpy as jnp
from jax import lax
from jax.experimental import pallas as pl
from jax.experimental.pallas import tpu as pltpu
from jax.experimental.pallas import tpu_sc as plsc

N = 10000
E = 320000
D_IN = 128
H = 8
HID = 8
OUT = 16
NEG_SLOPE = 0.2

C = 80
NCHUNK = E // C
NC, NS = 2, 16
NW = NC * NS
CPW = NCHUNK // NW
PAIRS = CPW // 2
ROWS_PER_TILE = N // NS


def _make_edge_pass(HF):
    F = HF // H
    mesh = plsc.VectorSubcoreMesh(core_axis_name="c", subcore_axis_name="s")

    @functools.partial(
        pl.kernel,
        out_type=[
            jax.ShapeDtypeStruct((NC, N, 16), jnp.float32),
            jax.ShapeDtypeStruct((NC, N, HF), jnp.float32),
        ],
        mesh=mesh,
        compiler_params=pltpu.CompilerParams(
            use_tc_tiling_on_sc=False, needs_layout_passes=False),
        scratch_types=[
            pltpu.VMEM((2, C), jnp.int32),
            pltpu.VMEM((2, C), jnp.int32),
            pltpu.VMEM((2, C, 16), jnp.float32),
            pltpu.VMEM((2, C, 16), jnp.float32),
            pltpu.VMEM((2, C, 16), jnp.float32),
            pltpu.VMEM((2, C, HF), jnp.bfloat16),
            pltpu.VMEM((2, C, HF), jnp.float32),
            pltpu.VMEM_SHARED((N, 16), jnp.float32),
            pltpu.VMEM_SHARED((N, HF), jnp.float32),
            pltpu.SemaphoreType.DMA,
            pltpu.SemaphoreType.DMA,
            pltpu.SemaphoreType.DMA,
            pltpu.SemaphoreType.DMA,
            pltpu.SemaphoreType.DMA,
            pltpu.SemaphoreType.DMA,
            pltpu.SemaphoreType.DMA,
            pltpu.SemaphoreType.DMA,
        ],
    )
    def edge_pass(el_hbm, er_hbm, feat_hbm, src_hbm, dst_hbm,
                  denom_out, acc_out,
                  sidx, didx, elg, erg, sv, fg, mg,
                  denom_s, acc_s,
                  sem_i0, sem_i1, sem_e0, sem_e1,
                  sem_f0, sem_f1, sem_s0, sem_s1):
        cid = lax.axis_index("c")
        sid = lax.axis_index("s")
        wid = cid * NS + sid
        base_c = wid * CPW
        zeros16 = jnp.zeros((16,), jnp.float32)
        sem_i = (sem_i0, sem_i1)
        sem_e = (sem_e0, sem_e1)
        sem_f = (sem_f0, sem_f1)
        sem_s = (sem_s0, sem_s1)

        def zrow(r, _):
            sv[0, r] = zeros16
            for k in range(HF // 16):
                mg[0, r, pl.ds(16 * k, 16)] = zeros16
            return 0
        lax.fori_loop(0, C, zrow, 0)
        base = sid * ROWS_PER_TILE
        for q in range(7):
            pltpu.sync_copy(sv.at[0],
                            denom_s.at[pl.ds(base + q * 80, 80)])
            pltpu.sync_copy(mg.at[0],
                            acc_s.at[pl.ds(base + q * 80, 80)])
        pltpu.sync_copy(sv.at[0, pl.ds(0, 65)],
                        denom_s.at[pl.ds(base + 560, 65)])
        pltpu.sync_copy(mg.at[0, pl.ds(0, 65)],
                        acc_s.at[pl.ds(base + 560, 65)])
        plsc.subcore_barrier()

        def issue_idx(b, c):
            pltpu.async_copy(src_hbm.at[c], sidx.at[b], sem_i[b])
            pltpu.async_copy(dst_hbm.at[c], didx.at[b], sem_i[b])

        def wait_idx(b):
            pltpu.make_async_copy(src_hbm.at[0], sidx.at[b], sem_i[b]).wait()
            pltpu.make_async_copy(src_hbm.at[0], didx.at[b], sem_i[b]).wait()

        def issue_gathers(b):
            pltpu.async_copy(el_hbm.at[sidx.at[b]], elg.at[b], sem_e[b])
            pltpu.async_copy(er_hbm.at[didx.at[b]], erg.at[b], sem_e[b])
            pltpu.async_copy(feat_hbm.at[sidx.at[b]], fg.at[b], sem_f[b])

        def wait_ee(b):
            pltpu.make_async_copy(
                el_hbm.at[sidx.at[b]], elg.at[b], sem_e[b]).wait()
            pltpu.make_async_copy(
                er_hbm.at[didx.at[b]], erg.at[b], sem_e[b]).wait()

        def wait_feat(b):
            pltpu.make_async_copy(
                feat_hbm.at[sidx.at[b]], fg.at[b], sem_f[b]).wait()

        def wait_scatters(b):
            pltpu.make_async_copy(
                sv.at[b], denom_s.at[didx.at[b]], sem_s[b]).wait()
            pltpu.make_async_copy(
                mg.at[b], acc_s.at[didx.at[b]], sem_s[b]).wait()

        def compute_and_scatter(b):
            def srow(rr, _):
                for dr in range(2):
                    r = 2 * rr + dr
                    e = elg[b, r] + erg[b, r]
                    sv[b, r] = jnp.exp(jnp.maximum(e, NEG_SLOPE * e))
                return 0
            lax.fori_loop(0, C // 2, srow, 0)
            pltpu.async_copy(sv.at[b], denom_s.at[didx.at[b]],
                             sem_s[b], add=True)
            wait_feat(b)

            def mrow(rr, _):
                mask8 = lax.iota(jnp.int32, 16) < 8
                for dr in range(2):
                    r = 2 * rr + dr
                    srow_v = sv[b, r]
                    for g in range(HF // 32):
                        fb = fg[b, r, pl.ds(32 * g, 32)]
                        lo, hi = plsc.unpack(
                            fb, format=plsc.PackFormat.INTERLEAVED)
                        if F == 16:
                            blo = jnp.broadcast_to(srow_v[2 * g], (16,))
                            bhi = jnp.broadcast_to(srow_v[2 * g + 1], (16,))
                        else:
                            s0 = jnp.broadcast_to(srow_v[4 * g], (16,))
                            s1 = jnp.broadcast_to(srow_v[4 * g + 1], (16,))
                            s2 = jnp.broadcast_to(srow_v[4 * g + 2], (16,))
                            s3 = jnp.broadcast_to(srow_v[4 * g + 3], (16,))
                            blo = jnp.where(mask8, s0, s1)
                            bhi = jnp.where(mask8, s2, s3)
                        mg[b, r, pl.ds(32 * g, 16)] = blo * lo
                        mg[b, r, pl.ds(32 * g + 16, 16)] = bhi * hi
                return 0
            lax.fori_loop(0, C // 2, mrow, 0)
            pltpu.async_copy(mg.at[b], acc_s.at[didx.at[b]],
                             sem_s[b], add=True)

        issue_idx(0, base_c)
        issue_idx(1, base_c + 1)
        wait_idx(0)
        issue_gathers(0)

        def pipe_body(j2, _):
            c0 = base_c + 2 * j2
            wait_ee(0)

            @pl.when(j2 > 0)
            def _():
                wait_scatters(1)
            wait_idx(1)
            issue_gathers(1)
            compute_and_scatter(0)

            issue_idx(0, c0 + 2)
            wait_ee(1)
            compute_and_scatter(1)

            @pl.when(j2 < PAIRS - 1)
            def _():
                issue_idx(1, c0 + 3)
            wait_scatters(0)

            wait_idx(0)
            issue_gathers(0)
            return 0

        lax.fori_loop(0, PAIRS, pipe_body, 0)
        wait_ee(0)
        compute_and_scatter(0)
        wait_scatters(0)
        wait_scatters(1)
        plsc.subcore_barrier()

        pltpu.sync_copy(denom_s.at[pl.ds(base, ROWS_PER_TILE)],
                        denom_out.at[cid, pl.ds(base, ROWS_PER_TILE)])
        pltpu.sync_copy(acc_s.at[pl.ds(base, ROWS_PER_TILE)],
                        acc_out.at[cid, pl.ds(base, ROWS_PER_TILE)])

    return edge_pass


_edge_pass_l1 = _make_edge_pass(H * HID)
_edge_pass_l2 = _make_edge_pass(H * OUT)


_BLK = 2000
_GRID = N // _BLK


def _mm(a, b):
    return jnp.dot(a, b, preferred_element_type=jnp.float32)


def _k1_body(x_ref, w_ref, al_ref, ar_ref, pm_ref, feat_ref, el_ref, er_ref):
    f = _mm(x_ref[...], w_ref[...])
    feat_ref[...] = _mm(f, pm_ref[...]).astype(jnp.bfloat16)
    el_ref[...] = _mm(f, al_ref[...])
    er_ref[...] = _mm(f, ar_ref[...])


def _k2_body(accp_ref, denp_ref, p1_ref, b1_ref, w2_ref, al_ref, ar_ref,
             pm_ref, feat_ref, el_ref, er_ref):
    acc = accp_ref[0] + accp_ref[1]
    den = denp_ref[0] + denp_ref[1]
    denb = _mm(den, p1_ref[...])
    out1 = acc / (denb + 1e-9) + b1_ref[...]
    hmat = jnp.where(out1 > 0, out1, jnp.exp(out1) - 1.0)
    f2 = _mm(hmat, w2_ref[...])
    feat_ref[...] = _mm(f2, pm_ref[...]).astype(jnp.bfloat16)
    el_ref[...] = _mm(f2, al_ref[...])
    er_ref[...] = _mm(f2, ar_ref[...])


def _k3_body(accp_ref, denp_ref, p2_ref, b2_ref, m_ref, out_ref):
    acc = accp_ref[0] + accp_ref[1]
    den = denp_ref[0] + denp_ref[1]
    denb = _mm(den, p2_ref[...])
    out2 = acc / (denb + 1e-9) + b2_ref[...]
    out_ref[...] = _mm(out2, m_ref[...])


def _row_spec(cols):
    return pl.BlockSpec((_BLK, cols), lambda i: (i, 0))


def _part_spec(cols):
    return pl.BlockSpec((NC, _BLK, cols), lambda i: (0, i, 0))


def _full_spec(rows, cols):
    return pl.BlockSpec((rows, cols), lambda i: (0, 0))


def _k1(x, w1, al1, ar1, pm1):
    HF1 = H * HID
    return pl.pallas_call(
        _k1_body,
        grid=(_GRID,),
        in_specs=[_row_spec(D_IN), _full_spec(D_IN, HF1),
                  _full_spec(HF1, 16), _full_spec(HF1, 16),
                  _full_spec(HF1, HF1)],
        out_specs=[_row_spec(HF1), _row_spec(16), _row_spec(16)],
        out_shape=[jax.ShapeDtypeStruct((N, HF1), jnp.bfloat16),
                   jax.ShapeDtypeStruct((N, 16), jnp.float32),
                   jax.ShapeDtypeStruct((N, 16), jnp.float32)],
    )(x, w1, al1, ar1, pm1)


def _k2(accp, denp, p1, b1row, w2, al2, ar2, pm2):
    HF1, HF2 = H * HID, H * OUT
    return pl.pallas_call(
        _k2_body,
        grid=(_GRID,),
        in_specs=[_part_spec(HF1), _part_spec(16), _full_spec(16, HF1),
                  _full_spec(1, HF1), _full_spec(HF1, HF2),
                  _full_spec(HF2, 16), _full_spec(HF2, 16),
                  _full_spec(HF2, HF2)],
        out_specs=[_row_spec(HF2), _row_spec(16), _row_spec(16)],
        out_shape=[jax.ShapeDtypeStruct((N, HF2), jnp.bfloat16),
                   jax.ShapeDtypeStruct((N, 16), jnp.float32),
                   jax.ShapeDtypeStruct((N, 16), jnp.float32)],
    )(accp, denp, p1, b1row, w2, al2, ar2, pm2)


def _k3(accp, denp, p2, b2row, m):
    HF2 = H * OUT
    return pl.pallas_call(
        _k3_body,
        grid=(_GRID,),
        in_specs=[_part_spec(HF2), _part_spec(16), _full_spec(16, HF2),
                  _full_spec(1, HF2), _full_spec(HF2, OUT)],
        out_specs=[_row_spec(OUT)],
        out_shape=[jax.ShapeDtypeStruct((N, OUT), jnp.float32)],
    )(accp, denp, p2, b2row, m)


def _block_diag_att(a, hf):
    k = jnp.arange(hf)
    return jnp.zeros((hf, 16), jnp.float32).at[k, k // (hf // H)].set(
        a.reshape(-1))


def _head_expand(hf):
    k = jnp.arange(hf)
    return jnp.zeros((16, hf), jnp.float32).at[k // (hf // H), k].set(1.0)


def _interleave_perm(hf):
    o = jnp.arange(hf)
    g, t = o // 32, o % 32
    new = jnp.where(t < 16, 32 * g + 2 * t, 32 * g + 2 * (t - 16) + 1)
    return jnp.zeros((hf, hf), jnp.float32).at[o, new].set(1.0)


@jax.jit
def kernel(node_feat, edge_index, W1, a_l1, a_r1, b1, W2, a_l2, a_r2, b2):
    src2d = edge_index[0].reshape(NCHUNK, C)
    dst2d = edge_index[1].reshape(NCHUNK, C)


    al1 = _block_diag_att(a_l1, H * HID)
    ar1 = _block_diag_att(a_r1, H * HID)
    al2 = _block_diag_att(a_l2, H * OUT)
    ar2 = _block_diag_att(a_r2, H * OUT)
    p1 = _head_expand(H * HID)
    p2 = _head_expand(H * OUT)
    karange = jnp.arange(H * OUT)
    m = jnp.zeros((H * OUT, OUT), jnp.float32).at[
        karange, karange % OUT].set(1.0 / H)

    pm1 = _interleave_perm(H * HID)
    pm2 = _interleave_perm(H * OUT)
    feat1, el1, er1 = _k1(node_feat, W1, al1, ar1, pm1)
    den1, acc1 = _edge_pass_l1(el1, er1, feat1, src2d, dst2d)
    feat2, el2, er2 = _k2(acc1, den1, p1, b1.reshape(1, -1), W2, al2, ar2, pm2)
    den2, acc2 = _edge_pass_l2(el2, er2, feat2, src2d, dst2d)
    (logits,) = _k3(acc2, den2, p2, b2.reshape(1, -1), m)
    return logits


# speedup vs baseline: 1.3876x; 1.3876x over previous
import functools
import jax
import jax.numpy as jnp
from jax import lax
from jax.experimental import pallas as pl
from jax.experimental.pallas import tpu as pltpu
from jax.experimental.pallas import tpu_sc as plsc

N = 10000
E = 320000
D_IN = 128
H = 8
HID = 8
OUT = 16
NEG_SLOPE = 0.2

C = 100
NCHUNK = E // C
NC, NS = 2, 16
NW = NC * NS
CPW = NCHUNK // NW
ROWS_PER_TILE = N // NS


def _make_edge_pass(HF):
    F = HF // H
    mesh = plsc.VectorSubcoreMesh(core_axis_name="c", subcore_axis_name="s")

    @functools.partial(
        pl.kernel,
        out_type=[
            jax.ShapeDtypeStruct((NC, N, 16), jnp.float32),
            jax.ShapeDtypeStruct((NC, N, HF), jnp.float32),
        ],
        mesh=mesh,
        compiler_params=pltpu.CompilerParams(
            use_tc_tiling_on_sc=False, needs_layout_passes=False),
        scratch_types=[
            pltpu.VMEM((2, C), jnp.int32),
            pltpu.VMEM((2, C), jnp.int32),
            pltpu.VMEM((2, C, 16), jnp.float32),
            pltpu.VMEM((2, C, 16), jnp.float32),
            pltpu.VMEM((2, C, 16), jnp.float32),
            pltpu.VMEM((2, C, HF), jnp.float32),
            pltpu.VMEM_SHARED((N, 16), jnp.float32),
            pltpu.VMEM_SHARED((N, HF), jnp.float32),
            pltpu.SemaphoreType.DMA,
            pltpu.SemaphoreType.DMA,
            pltpu.SemaphoreType.DMA,
            pltpu.SemaphoreType.DMA,
            pltpu.SemaphoreType.DMA,
            pltpu.SemaphoreType.DMA,
            pltpu.SemaphoreType.DMA,
            pltpu.SemaphoreType.DMA,
        ],
    )
    def edge_pass(el_hbm, er_hbm, feat_hbm, src_hbm, dst_hbm,
                  denom_out, acc_out,
                  sidx, didx, elg, erg, sv, fg,
                  denom_s, acc_s,
                  sem_i0, sem_i1, sem_e0, sem_e1,
                  sem_f0, sem_f1, sem_s0, sem_s1):
        cid = lax.axis_index("c")
        sid = lax.axis_index("s")
        wid = cid * NS + sid
        base_c = wid * CPW
        zeros16 = jnp.zeros((16,), jnp.float32)
        sem_i = (sem_i0, sem_i1)
        sem_e = (sem_e0, sem_e1)
        sem_f = (sem_f0, sem_f1)
        sem_s = (sem_s0, sem_s1)

        def issue_idx(b, c):
            pltpu.async_copy(src_hbm.at[c], sidx.at[b], sem_i[b])
            pltpu.async_copy(dst_hbm.at[c], didx.at[b], sem_i[b])

        def wait_idx(b):
            pltpu.make_async_copy(src_hbm.at[0], sidx.at[b], sem_i[b]).wait()
            pltpu.make_async_copy(src_hbm.at[0], didx.at[b], sem_i[b]).wait()

        def issue_gathers(b):
            pltpu.async_copy(el_hbm.at[sidx.at[b]], elg.at[b], sem_e[b])
            pltpu.async_copy(er_hbm.at[didx.at[b]], erg.at[b], sem_e[b])
            pltpu.async_copy(feat_hbm.at[sidx.at[b]], fg.at[b], sem_f[b])

        issue_idx(0, base_c)
        issue_idx(1, base_c + 1)
        wait_idx(0)
        issue_gathers(0)

        def zrow(r, _):
            sv[1, r] = zeros16
            for k in range(HF // 16):
                fg[1, r, pl.ds(16 * k, 16)] = zeros16
            return 0
        lax.fori_loop(0, C, zrow, 0)
        base = sid * ROWS_PER_TILE
        for q in range(6):
            pltpu.sync_copy(sv.at[1],
                            denom_s.at[pl.ds(base + q * 100, 100)])
            pltpu.sync_copy(fg.at[1],
                            acc_s.at[pl.ds(base + q * 100, 100)])
        pltpu.sync_copy(sv.at[1, pl.ds(0, 25)],
                        denom_s.at[pl.ds(base + 600, 25)])
        pltpu.sync_copy(fg.at[1, pl.ds(0, 25)],
                        acc_s.at[pl.ds(base + 600, 25)])
        plsc.subcore_barrier()

        def wait_ee(b):
            pltpu.make_async_copy(
                el_hbm.at[sidx.at[b]], elg.at[b], sem_e[b]).wait()
            pltpu.make_async_copy(
                er_hbm.at[didx.at[b]], erg.at[b], sem_e[b]).wait()

        def wait_feat(b):
            pltpu.make_async_copy(
                feat_hbm.at[sidx.at[b]], fg.at[b], sem_f[b]).wait()

        def wait_scatters(b):
            pltpu.make_async_copy(
                sv.at[b], denom_s.at[didx.at[b]], sem_s[b]).wait()
            pltpu.make_async_copy(
                fg.at[b], acc_s.at[didx.at[b]], sem_s[b]).wait()

        def compute_and_scatter(b):
            def srow(rr, _):
                for dr in range(2):
                    r = 2 * rr + dr
                    e = elg[b, r] + erg[b, r]
                    sv[b, r] = jnp.exp(jnp.maximum(e, NEG_SLOPE * e))
                return 0
            lax.fori_loop(0, C // 2, srow, 0)
            pltpu.async_copy(sv.at[b], denom_s.at[didx.at[b]],
                             sem_s[b], add=True)
            wait_feat(b)

            def mrow(rr, _):
                mask8 = lax.iota(jnp.int32, 16) < 8
                for dr in range(2):
                    r = 2 * rr + dr
                    srow_v = sv[b, r]
                    for k in range(HF // 16):
                        if F == 16:
                            bval = jnp.broadcast_to(srow_v[k], (16,))
                        else:
                            b0 = jnp.broadcast_to(srow_v[2 * k], (16,))
                            b1 = jnp.broadcast_to(srow_v[2 * k + 1], (16,))
                            bval = jnp.where(mask8, b0, b1)
                        fg[b, r, pl.ds(16 * k, 16)] = (
                            bval * fg[b, r, pl.ds(16 * k, 16)])
                return 0
            lax.fori_loop(0, C // 2, mrow, 0)
            pltpu.async_copy(fg.at[b], acc_s.at[didx.at[b]],
                             sem_s[b], add=True)

        def pipe_body(j2, _):
            c0 = base_c + 2 * j2
            wait_ee(0)

            @pl.when(j2 > 0)
            def _():
                wait_scatters(1)
            wait_idx(1)
            issue_gathers(1)
            compute_and_scatter(0)

            @pl.when(j2 < CPW // 2 - 1)
            def _():
                issue_idx(0, c0 + 2)
            wait_ee(1)
            compute_and_scatter(1)

            @pl.when(j2 < CPW // 2 - 1)
            def _():
                issue_idx(1, c0 + 3)
            wait_scatters(0)

            @pl.when(j2 < CPW // 2 - 1)
            def _():
                wait_idx(0)
                issue_gathers(0)
            return 0

        lax.fori_loop(0, CPW // 2, pipe_body, 0)
        wait_scatters(1)
        plsc.subcore_barrier()

        pltpu.sync_copy(denom_s.at[pl.ds(base, ROWS_PER_TILE)],
                        denom_out.at[cid, pl.ds(base, ROWS_PER_TILE)])
        pltpu.sync_copy(acc_s.at[pl.ds(base, ROWS_PER_TILE)],
                        acc_out.at[cid, pl.ds(base, ROWS_PER_TILE)])

    return edge_pass


_edge_pass_l1 = _make_edge_pass(H * HID)
_edge_pass_l2 = _make_edge_pass(H * OUT)


_BLK = 2000
_GRID = N // _BLK


def _mm(a, b):
    return jnp.dot(a, b, preferred_element_type=jnp.float32)


def _k1_body(x_ref, w_ref, al_ref, ar_ref, feat_ref, el_ref, er_ref):
    f = _mm(x_ref[...], w_ref[...])
    feat_ref[...] = f
    el_ref[...] = _mm(f, al_ref[...])
    er_ref[...] = _mm(f, ar_ref[...])


def _k2_body(accp_ref, denp_ref, p1_ref, b1_ref, w2_ref, al_ref, ar_ref,
             feat_ref, el_ref, er_ref):
    acc = accp_ref[0] + accp_ref[1]
    den = denp_ref[0] + denp_ref[1]
    denb = _mm(den, p1_ref[...])
    out1 = acc / (denb + 1e-9) + b1_ref[...]
    hmat = jnp.where(out1 > 0, out1, jnp.exp(out1) - 1.0)
    f2 = _mm(hmat, w2_ref[...])
    feat_ref[...] = f2
    el_ref[...] = _mm(f2, al_ref[...])
    er_ref[...] = _mm(f2, ar_ref[...])


def _k3_body(accp_ref, denp_ref, p2_ref, b2_ref, m_ref, out_ref):
    acc = accp_ref[0] + accp_ref[1]
    den = denp_ref[0] + denp_ref[1]
    denb = _mm(den, p2_ref[...])
    out2 = acc / (denb + 1e-9) + b2_ref[...]
    out_ref[...] = _mm(out2, m_ref[...])


def _row_spec(cols):
    return pl.BlockSpec((_BLK, cols), lambda i: (i, 0))


def _part_spec(cols):
    return pl.BlockSpec((NC, _BLK, cols), lambda i: (0, i, 0))


def _full_spec(rows, cols):
    return pl.BlockSpec((rows, cols), lambda i: (0, 0))


def _k1(x, w1, al1, ar1):
    return pl.pallas_call(
        _k1_body,
        grid=(_GRID,),
        in_specs=[_row_spec(D_IN), _full_spec(D_IN, H * HID),
                  _full_spec(H * HID, 16), _full_spec(H * HID, 16)],
        out_specs=[_row_spec(H * HID), _row_spec(16), _row_spec(16)],
        out_shape=[jax.ShapeDtypeStruct((N, H * HID), jnp.float32),
                   jax.ShapeDtypeStruct((N, 16), jnp.float32),
                   jax.ShapeDtypeStruct((N, 16), jnp.float32)],
    )(x, w1, al1, ar1)


def _k2(accp, denp, p1, b1row, w2, al2, ar2):
    HF1, HF2 = H * HID, H * OUT
    return pl.pallas_call(
        _k2_body,
        grid=(_GRID,),
        in_specs=[_part_spec(HF1), _part_spec(16), _full_spec(16, HF1),
                  _full_spec(1, HF1), _full_spec(HF1, HF2),
                  _full_spec(HF2, 16), _full_spec(HF2, 16)],
        out_specs=[_row_spec(HF2), _row_spec(16), _row_spec(16)],
        out_shape=[jax.ShapeDtypeStruct((N, HF2), jnp.float32),
                   jax.ShapeDtypeStruct((N, 16), jnp.float32),
                   jax.ShapeDtypeStruct((N, 16), jnp.float32)],
    )(accp, denp, p1, b1row, w2, al2, ar2)


def _k3(accp, denp, p2, b2row, m):
    HF2 = H * OUT
    return pl.pallas_call(
        _k3_body,
        grid=(_GRID,),
        in_specs=[_part_spec(HF2), _part_spec(16), _full_spec(16, HF2),
                  _full_spec(1, HF2), _full_spec(HF2, OUT)],
        out_specs=[_row_spec(OUT)],
        out_shape=[jax.ShapeDtypeStruct((N, OUT), jnp.float32)],
    )(accp, denp, p2, b2row, m)


def _block_diag_att(a, hf):
    k = jnp.arange(hf)
    return jnp.zeros((hf, 16), jnp.float32).at[k, k // (hf // H)].set(
        a.reshape(-1))


def _head_expand(hf):
    k = jnp.arange(hf)
    return jnp.zeros((16, hf), jnp.float32).at[k // (hf // H), k].set(1.0)


@jax.jit
def kernel(node_feat, edge_index, W1, a_l1, a_r1, b1, W2, a_l2, a_r2, b2):
    src2d = edge_index[0].reshape(NCHUNK, C)
    dst2d = edge_index[1].reshape(NCHUNK, C)


    al1 = _block_diag_att(a_l1, H * HID)
    ar1 = _block_diag_att(a_r1, H * HID)
    al2 = _block_diag_att(a_l2, H * OUT)
    ar2 = _block_diag_att(a_r2, H * OUT)
    p1 = _head_expand(H * HID)
    p2 = _head_expand(H * OUT)
    karange = jnp.arange(H * OUT)
    m = jnp.zeros((H * OUT, OUT), jnp.float32).at[
        karange, karange % OUT].set(1.0 / H)

    feat1, el1, er1 = _k1(node_feat, W1, al1, ar1)
    den1, acc1 = _edge_pass_l1(el1, er1, feat1, src2d, dst2d)
    feat2, el2, er2 = _k2(acc1, den1, p1, b1.reshape(1, -1), W2, al2, ar2)
    den2, acc2 = _edge_pass_l2(el2, er2, feat2, src2d, dst2d)
    (logits,) = _k3(acc2, den2, p2, b2.reshape(1, -1), m)
    return logits


# speedup vs baseline: 1.4262x; 1.0278x over previous
import functools
import jax
import jax.numpy as jnp
from jax import lax
from jax.experimental import pallas as pl
from jax.experimental.pallas import tpu as pltpu
from jax.experimental.pallas import tpu_sc as plsc

N = 10000
E = 320000
D_IN = 128
H = 8
HID = 8
OUT = 16
NEG_SLOPE = 0.2

C = 100
NCHUNK = E // C
NC, NS = 2, 16
NW = NC * NS
CPW = NCHUNK // NW
ROWS_PER_TILE = N // NS


def _make_edge_pass(HF):
    F = HF // H
    mesh = plsc.VectorSubcoreMesh(core_axis_name="c", subcore_axis_name="s")

    @functools.partial(
        pl.kernel,
        out_type=[
            jax.ShapeDtypeStruct((NC, N, 16), jnp.float32),
            jax.ShapeDtypeStruct((NC, N, HF), jnp.float32),
        ],
        mesh=mesh,
        compiler_params=pltpu.CompilerParams(
            use_tc_tiling_on_sc=False, needs_layout_passes=False),
        scratch_types=[
            pltpu.VMEM((2, C), jnp.int32),
            pltpu.VMEM((2, C), jnp.int32),
            pltpu.VMEM((2, C, 16), jnp.float32),
            pltpu.VMEM((2, C, 16), jnp.float32),
            pltpu.VMEM((2, C, 16), jnp.float32),
            pltpu.VMEM((2, C, HF), jnp.float32),
            pltpu.VMEM_SHARED((N, 16), jnp.float32),
            pltpu.VMEM_SHARED((N, HF), jnp.float32),
            pltpu.SemaphoreType.DMA,
            pltpu.SemaphoreType.DMA,
            pltpu.SemaphoreType.DMA,
            pltpu.SemaphoreType.DMA,
            pltpu.SemaphoreType.DMA,
            pltpu.SemaphoreType.DMA,
            pltpu.SemaphoreType.DMA,
            pltpu.SemaphoreType.DMA,
        ],
    )
    def edge_pass(el_hbm, er_hbm, feat_hbm, src_hbm, dst_hbm,
                  denom_out, acc_out,
                  sidx, didx, elg, erg, sv, fg,
                  denom_s, acc_s,
                  sem_i0, sem_i1, sem_e0, sem_e1,
                  sem_f0, sem_f1, sem_s0, sem_s1):
        cid = lax.axis_index("c")
        sid = lax.axis_index("s")
        wid = cid * NS + sid
        base_c = wid * CPW
        zeros16 = jnp.zeros((16,), jnp.float32)
        sem_i = (sem_i0, sem_i1)
        sem_e = (sem_e0, sem_e1)
        sem_f = (sem_f0, sem_f1)
        sem_s = (sem_s0, sem_s1)

        def issue_idx(b, c):
            pltpu.async_copy(src_hbm.at[c], sidx.at[b], sem_i[b])
            pltpu.async_copy(dst_hbm.at[c], didx.at[b], sem_i[b])

        def wait_idx(b):
            pltpu.make_async_copy(src_hbm.at[0], sidx.at[b], sem_i[b]).wait()
            pltpu.make_async_copy(src_hbm.at[0], didx.at[b], sem_i[b]).wait()

        def issue_gathers(b):
            pltpu.async_copy(el_hbm.at[sidx.at[b]], elg.at[b], sem_e[b])
            pltpu.async_copy(er_hbm.at[didx.at[b]], erg.at[b], sem_e[b])
            pltpu.async_copy(feat_hbm.at[sidx.at[b]], fg.at[b], sem_f[b])

        issue_idx(0, base_c)
        issue_idx(1, base_c + 1)
        wait_idx(0)
        issue_gathers(0)

        def zrow(r, _):
            sv[1, r] = zeros16
            for k in range(HF // 16):
                fg[1, r, pl.ds(16 * k, 16)] = zeros16
            return 0
        lax.fori_loop(0, C, zrow, 0)
        base = sid * ROWS_PER_TILE
        for q in range(6):
            pltpu.sync_copy(sv.at[1],
                            denom_s.at[pl.ds(base + q * 100, 100)])
            pltpu.sync_copy(fg.at[1],
                            acc_s.at[pl.ds(base + q * 100, 100)])
        pltpu.sync_copy(sv.at[1, pl.ds(0, 25)],
                        denom_s.at[pl.ds(base + 600, 25)])
        pltpu.sync_copy(fg.at[1, pl.ds(0, 25)],
                        acc_s.at[pl.ds(base + 600, 25)])
        plsc.subcore_barrier()

        def wait_ee(b):
            pltpu.make_async_copy(
                el_hbm.at[sidx.at[b]], elg.at[b], sem_e[b]).wait()
            pltpu.make_async_copy(
                er_hbm.at[didx.at[b]], erg.at[b], sem_e[b]).wait()

        def wait_feat(b):
            pltpu.make_async_copy(
                feat_hbm.at[sidx.at[b]], fg.at[b], sem_f[b]).wait()

        def wait_scatters(b):
            pltpu.make_async_copy(
                sv.at[b], denom_s.at[didx.at[b]], sem_s[b]).wait()
            pltpu.make_async_copy(
                fg.at[b], acc_s.at[didx.at[b]], sem_s[b]).wait()

        def compute_and_scatter(b):
            def srow(rr, _):
                for dr in range(4):
                    r = 4 * rr + dr
                    e = elg[b, r] + erg[b, r]
                    sv[b, r] = jnp.exp(jnp.maximum(e, NEG_SLOPE * e))
                return 0
            lax.fori_loop(0, C // 4, srow, 0)
            pltpu.async_copy(sv.at[b], denom_s.at[didx.at[b]],
                             sem_s[b], add=True)
            wait_feat(b)

            def mrow(rr, _):
                mask8 = lax.iota(jnp.int32, 16) < 8
                for dr in range(4):
                    r = 4 * rr + dr
                    srow_v = sv[b, r]
                    for k in range(HF // 16):
                        if F == 16:
                            bval = jnp.broadcast_to(srow_v[k], (16,))
                        else:
                            b0 = jnp.broadcast_to(srow_v[2 * k], (16,))
                            b1 = jnp.broadcast_to(srow_v[2 * k + 1], (16,))
                            bval = jnp.where(mask8, b0, b1)
                        fg[b, r, pl.ds(16 * k, 16)] = (
                            bval * fg[b, r, pl.ds(16 * k, 16)])
                return 0
            lax.fori_loop(0, C // 4, mrow, 0)
            pltpu.async_copy(fg.at[b], acc_s.at[didx.at[b]],
                             sem_s[b], add=True)

        def pipe_body(j2, _):
            c0 = base_c + 2 * j2
            wait_ee(0)

            @pl.when(j2 > 0)
            def _():
                wait_scatters(1)
            wait_idx(1)
            issue_gathers(1)
            compute_and_scatter(0)

            @pl.when(j2 < CPW // 2 - 1)
            def _():
                issue_idx(0, c0 + 2)
            wait_ee(1)
            compute_and_scatter(1)

            @pl.when(j2 < CPW // 2 - 1)
            def _():
                issue_idx(1, c0 + 3)
            wait_scatters(0)

            @pl.when(j2 < CPW // 2 - 1)
            def _():
                wait_idx(0)
                issue_gathers(0)
            return 0

        lax.fori_loop(0, CPW // 2, pipe_body, 0)
        wait_scatters(1)
        plsc.subcore_barrier()

        pltpu.sync_copy(denom_s.at[pl.ds(base, ROWS_PER_TILE)],
                        denom_out.at[cid, pl.ds(base, ROWS_PER_TILE)])
        pltpu.sync_copy(acc_s.at[pl.ds(base, ROWS_PER_TILE)],
                        acc_out.at[cid, pl.ds(base, ROWS_PER_TILE)])

    return edge_pass


_edge_pass_l1 = _make_edge_pass(H * HID)
_edge_pass_l2 = _make_edge_pass(H * OUT)


_BLK = 2000
_GRID = N // _BLK


def _mm(a, b):
    return jnp.dot(a, b, preferred_element_type=jnp.float32)


def _k1_body(x_ref, w_ref, al_ref, ar_ref, feat_ref, el_ref, er_ref):
    f = _mm(x_ref[...], w_ref[...])
    feat_ref[...] = f
    el_ref[...] = _mm(f, al_ref[...])
    er_ref[...] = _mm(f, ar_ref[...])


def _k2_body(accp_ref, denp_ref, p1_ref, b1_ref, w2_ref, al_ref, ar_ref,
             feat_ref, el_ref, er_ref):
    acc = accp_ref[0] + accp_ref[1]
    den = denp_ref[0] + denp_ref[1]
    denb = _mm(den, p1_ref[...])
    out1 = acc / (denb + 1e-9) + b1_ref[...]
    hmat = jnp.where(out1 > 0, out1, jnp.exp(out1) - 1.0)
    f2 = _mm(hmat, w2_ref[...])
    feat_ref[...] = f2
    el_ref[...] = _mm(f2, al_ref[...])
    er_ref[...] = _mm(f2, ar_ref[...])


def _k3_body(accp_ref, denp_ref, p2_ref, b2_ref, m_ref, out_ref):
    acc = accp_ref[0] + accp_ref[1]
    den = denp_ref[0] + denp_ref[1]
    denb = _mm(den, p2_ref[...])
    out2 = acc / (denb + 1e-9) + b2_ref[...]
    out_ref[...] = _mm(out2, m_ref[...])


def _row_spec(cols):
    return pl.BlockSpec((_BLK, cols), lambda i: (i, 0))


def _part_spec(cols):
    return pl.BlockSpec((NC, _BLK, cols), lambda i: (0, i, 0))


def _full_spec(rows, cols):
    return pl.BlockSpec((rows, cols), lambda i: (0, 0))


def _k1(x, w1, al1, ar1):
    return pl.pallas_call(
        _k1_body,
        grid=(_GRID,),
        in_specs=[_row_spec(D_IN), _full_spec(D_IN, H * HID),
                  _full_spec(H * HID, 16), _full_spec(H * HID, 16)],
        out_specs=[_row_spec(H * HID), _row_spec(16), _row_spec(16)],
        out_shape=[jax.ShapeDtypeStruct((N, H * HID), jnp.float32),
                   jax.ShapeDtypeStruct((N, 16), jnp.float32),
                   jax.ShapeDtypeStruct((N, 16), jnp.float32)],
    )(x, w1, al1, ar1)


def _k2(accp, denp, p1, b1row, w2, al2, ar2):
    HF1, HF2 = H * HID, H * OUT
    return pl.pallas_call(
        _k2_body,
        grid=(_GRID,),
        in_specs=[_part_spec(HF1), _part_spec(16), _full_spec(16, HF1),
                  _full_spec(1, HF1), _full_spec(HF1, HF2),
                  _full_spec(HF2, 16), _full_spec(HF2, 16)],
        out_specs=[_row_spec(HF2), _row_spec(16), _row_spec(16)],
        out_shape=[jax.ShapeDtypeStruct((N, HF2), jnp.float32),
                   jax.ShapeDtypeStruct((N, 16), jnp.float32),
                   jax.ShapeDtypeStruct((N, 16), jnp.float32)],
    )(accp, denp, p1, b1row, w2, al2, ar2)


def _k3(accp, denp, p2, b2row, m):
    HF2 = H * OUT
    return pl.pallas_call(
        _k3_body,
        grid=(_GRID,),
        in_specs=[_part_spec(HF2), _part_spec(16), _full_spec(16, HF2),
                  _full_spec(1, HF2), _full_spec(HF2, OUT)],
        out_specs=[_row_spec(OUT)],
        out_shape=[jax.ShapeDtypeStruct((N, OUT), jnp.float32)],
    )(accp, denp, p2, b2row, m)


def _block_diag_att(a, hf):
    k = jnp.arange(hf)
    return jnp.zeros((hf, 16), jnp.float32).at[k, k // (hf // H)].set(
        a.reshape(-1))


def _head_expand(hf):
    k = jnp.arange(hf)
    return jnp.zeros((16, hf), jnp.float32).at[k // (hf // H), k].set(1.0)


@jax.jit
def kernel(node_feat, edge_index, W1, a_l1, a_r1, b1, W2, a_l2, a_r2, b2):
    src2d = edge_index[0].reshape(NCHUNK, C)
    dst2d = edge_index[1].reshape(NCHUNK, C)


    al1 = _block_diag_att(a_l1, H * HID)
    ar1 = _block_diag_att(a_r1, H * HID)
    al2 = _block_diag_att(a_l2, H * OUT)
    ar2 = _block_diag_att(a_r2, H * OUT)
    p1 = _head_expand(H * HID)
    p2 = _head_expand(H * OUT)
    karange = jnp.arange(H * OUT)
    m = jnp.zeros((H * OUT, OUT), jnp.float32).at[
        karange, karange % OUT].set(1.0 / H)

    feat1, el1, er1 = _k1(node_feat, W1, al1, ar1)
    den1, acc1 = _edge_pass_l1(el1, er1, feat1, src2d, dst2d)
    feat2, el2, er2 = _k2(acc1, den1, p1, b1.reshape(1, -1), W2, al2, ar2)
    den2, acc2 = _edge_pass_l2(el2, er2, feat2, src2d, dst2d)
    (logits,) = _k3(acc2, den2, p2, b2.reshape(1, -1), m)
    return logits
